# baseline (speedup 1.0000x reference)
import functools

import jax
import jax.numpy as jnp
from jax import lax
from jax.experimental import pallas as pl
from jax.experimental.pallas import tpu as pltpu
from jax.experimental.pallas import tpu_sc as plsc

N_FIELDS = 26
CARD = 100000
DIM = 24
BATCH = 16384

NC = 2
NS = 16
NW = NC * NS
ROWS = BATCH * N_FIELDS
ROWS_W = ROWS // NW
PERIOD = 208
CHUNK = 128
GROUP = 1664
CHUNKS_PER_GROUP = GROUP // CHUNK
GROUPS = ROWS_W // GROUP

assert ROWS_W % PERIOD == 0 and ROWS_W % N_FIELDS == 0
assert ROWS_W % GROUP == 0 and GROUP % CHUNK == 0

_mesh = plsc.VectorSubcoreMesh(core_axis_name="c", subcore_axis_name="s")


@functools.partial(
    pl.kernel,
    mesh=_mesh,
    compiler_params=pltpu.CompilerParams(use_tc_tiling_on_sc=False),
    out_type=jax.ShapeDtypeStruct((ROWS, DIM), jnp.float32),
    scratch_types=[
        pltpu.VMEM((ROWS_W,), jnp.int32),
        pltpu.VMEM((PERIOD,), jnp.int32),
        pltpu.VMEM((GROUP, DIM), jnp.float32),
        pltpu.SemaphoreType.DMA,
    ],
)
def _embed_gather(x_ref, table_ref, out_ref, idx_v, offs_v, rows_v, gsem):
    wid = lax.axis_index("s") * NC + lax.axis_index("c")
    base = pl.multiple_of(wid * ROWS_W, ROWS_W)

    pltpu.sync_copy(x_ref.at[pl.ds(base, ROWS_W)], idx_v)

    for j in range(PERIOD // 16):
        pos = j * 16 + lax.iota(jnp.int32, 16)
        offs_v[pl.ds(j * 16, 16)] = lax.rem(pos, N_FIELDS) * CARD

    def add_offsets(g, _):
        b = pl.multiple_of(g * PERIOD, PERIOD)
        for j in range(PERIOD // 16):
            s = b + j * 16
            idx_v[pl.ds(s, 16)] = idx_v[pl.ds(s, 16)] + offs_v[pl.ds(j * 16, 16)]
        return _

    lax.fori_loop(0, ROWS_W // PERIOD, add_offsets, None)

    def do_group(g, _):
        g0 = pl.multiple_of(g * GROUP, GROUP)
        copies = []
        for c in range(CHUNKS_PER_GROUP):
            o = g0 + c * CHUNK
            copies.append(
                pltpu.async_copy(
                    table_ref.at[idx_v.at[pl.ds(o, CHUNK)]],
                    rows_v.at[pl.ds(c * CHUNK, CHUNK)],
                    gsem,
                )
            )
        for cp in copies:
            cp.wait()
        pltpu.sync_copy(rows_v, out_ref.at[pl.ds(base + g0, GROUP)])
        return _

    lax.fori_loop(0, GROUPS, do_group, None)


def kernel(x_cat, tables):
    x_flat = x_cat.reshape(ROWS)
    table2d = tables.reshape(N_FIELDS * CARD, DIM)
    out = _embed_gather(x_flat, table2d)
    return out.reshape(BATCH, N_FIELDS * DIM)


# speedup vs baseline: 2.0737x; 2.0737x over previous
import functools

import jax
import jax.numpy as jnp
from jax import lax
from jax.experimental import pallas as pl
from jax.experimental.pallas import tpu as pltpu
from jax.experimental.pallas import tpu_sc as plsc

N_FIELDS = 26
CARD = 100000
DIM = 24
BATCH = 16384
OUT_W = N_FIELDS * DIM

NC = 2
NS = 16
NW = NC * NS
ROWS_B = BATCH // NW
IDX_W = ROWS_B * N_FIELDS
BLK = 16
NBLK = ROWS_B // BLK
BLK_IDX = BLK * N_FIELDS
BLK_F = BLK * OUT_W

_mesh = plsc.VectorSubcoreMesh(core_axis_name="c", subcore_axis_name="s")


@functools.partial(
    pl.kernel,
    mesh=_mesh,
    out_type=jax.ShapeDtypeStruct((BATCH * N_FIELDS, DIM), jnp.float32),
    scratch_types=[
        pltpu.VMEM((IDX_W,), jnp.int32),
        pltpu.VMEM((2, BLK_IDX, DIM), jnp.float32),
        pltpu.SemaphoreType.DMA,
        pltpu.SemaphoreType.DMA,
    ],
)
def _embed_gather(x_ref, table_ref, out_ref, idx_v, buf_v, gsem, osem):
    wid = lax.axis_index("s") * NC + lax.axis_index("c")
    ibase = pl.multiple_of(wid * IDX_W, IDX_W)

    pltpu.sync_copy(x_ref.at[pl.ds(ibase, IDX_W)], idx_v)

    def add_offsets(g, _):
        b = pl.multiple_of(g * 208, 208)
        for j in range(13):
            pos = j * 16 + lax.iota(jnp.int32, 16)
            off = lax.rem(pos, N_FIELDS) * CARD
            idx_v[pl.ds(b + j * 16, 16)] = idx_v[pl.ds(b + j * 16, 16)] + off
        return _

    lax.fori_loop(0, IDX_W // 208, add_offsets, None)

    def do_block(blk, _):
        buf = buf_v.at[lax.rem(blk, 2)]
        @pl.when(blk >= 2)
        def _wait_store():
            pltpu.make_async_copy(
                out_ref.at[pl.ds(0, BLK_IDX), :], buf, osem
            ).wait()

        iblk = pl.multiple_of(blk * BLK_IDX, BLK_IDX)

        def fire_grp(g, _):
            vec = idx_v[pl.ds(pl.multiple_of(iblk + g * 16, 16), 16)]
            d = g * 16
            for l in range(16):
                row = vec[l]
                pltpu.async_copy(
                    table_ref.at[pl.ds(row, 1), :],
                    buf.at[pl.ds(d + l, 1), :],
                    gsem,
                )
            return _

        lax.fori_loop(0, BLK_IDX // 16, fire_grp, None)
        pltpu.make_async_copy(out_ref.at[pl.ds(0, BLK_IDX), :], buf, gsem).wait()
        o = pl.multiple_of(ibase + blk * BLK_IDX, 8)
        pltpu.make_async_copy(buf, out_ref.at[pl.ds(o, BLK_IDX), :], osem).start()
        return _

    lax.fori_loop(0, NBLK, do_block, None)
    pltpu.make_async_copy(
        out_ref.at[pl.ds(0, BLK_IDX), :], buf_v.at[0], osem
    ).wait()
    pltpu.make_async_copy(
        out_ref.at[pl.ds(0, BLK_IDX), :], buf_v.at[1], osem
    ).wait()


def kernel(x_cat, tables):
    x_flat = x_cat.reshape(BATCH * N_FIELDS)
    table2d = tables.reshape(N_FIELDS * CARD, DIM)
    out = _embed_gather(x_flat, table2d)
    return out.reshape(BATCH, OUT_W)
